# baseline (speedup 1.0000x reference)
import functools
import jax
import jax.numpy as jnp
from jax import lax
from jax.experimental import pallas as pl
from jax.experimental.pallas import tpu as pltpu
from jax.experimental.pallas import tpu_sc as plsc

_K = 16


def _batcher_pairs(n):
    pairs = []

    def merge(lo, m, r):
        step = r * 2
        if step < m:
            merge(lo, m, step)
            merge(lo + r, m, step)
            for i in range(lo + r, lo + m - r, step):
                pairs.append((i, i + r))
        else:
            pairs.append((lo, lo + r))

    def sort(lo, m):
        if m > 1:
            h = m // 2
            sort(lo, h)
            sort(lo + h, h)
            merge(lo, m, 1)

    sort(0, n)
    return pairs


_SORT16 = _batcher_pairs(_K)


def _sort_desc(v):
    v = list(v)
    for i, j in _SORT16:
        hi = jnp.maximum(v[i], v[j])
        lo = jnp.minimum(v[i], v[j])
        v[i], v[j] = hi, lo
    return v


def _bitonic_resort(v):
    n = len(v)
    if n == 1:
        return v
    h = n // 2
    hi = [jnp.maximum(v[i], v[i + h]) for i in range(h)]
    lo = [jnp.minimum(v[i], v[i + h]) for i in range(h)]
    return _bitonic_resort(hi) + _bitonic_resort(lo)


def _merge_top16(a, b):
    merged = [jnp.maximum(a[i], b[_K - 1 - i]) for i in range(_K)]
    return _bitonic_resort(merged)


def kernel(similarity_matrix):
    B, P, C = similarity_matrix.shape
    L = 16
    NW = 32
    CB = 128
    PH = P // 2
    NG = CB // L
    n_tasks = B * (C // CB)
    tasks_per_w = n_tasks // NW

    mesh = plsc.VectorSubcoreMesh(core_axis_name="c", subcore_axis_name="s")

    @functools.partial(
        pl.kernel,
        mesh=mesh,
        out_type=[
            jax.ShapeDtypeStruct((B, 1, C), jnp.float32),
            jax.ShapeDtypeStruct((B, P, C), jnp.float32),
        ],
        scratch_types=[
            pltpu.VMEM((PH, CB), jnp.float32),
            pltpu.VMEM((NG, _K, L), jnp.float32),
            pltpu.VMEM((1, CB), jnp.float32),
        ],
    )
    def sc_body(sim_hbm, scores_hbm, mask_hbm, in_v, run_v, sc_v):
        cid = lax.axis_index("c")
        sid = lax.axis_index("s")
        wid = sid * 2 + cid

        def task_body(t, carry):
            tid = t * NW + wid
            b = tid // (C // CB)
            cb = pl.multiple_of((tid % (C // CB)) * CB, CB)

            for h in (0, 1):
                pltpu.sync_copy(
                    sim_hbm.at[b, pl.ds(h * PH, PH), pl.ds(cb, CB)], in_v
                )
                for g in range(NG):
                    gs = pl.ds(g * L, L)
                    if h == 0:
                        neg = jnp.full((L,), -jnp.inf, dtype=jnp.float32)
                        run0 = tuple(neg for _ in range(_K))
                    else:
                        run0 = tuple(run_v[g, i, :] for i in range(_K))

                    def chunk_body(k, run):
                        tiles = [in_v[k * L + i, gs] for i in range(L)]
                        tiles = _sort_desc(tiles)
                        return tuple(_merge_top16(list(run), tiles))

                    run = lax.fori_loop(0, PH // L, chunk_body, run0)
                    for i in range(_K):
                        run_v[g, i, :] = run[i]

            for g in range(NG):
                sc_v[0, pl.ds(g * L, L)] = run_v[g, 0, :]
            pltpu.sync_copy(sc_v, scores_hbm.at[b, :, pl.ds(cb, CB)])

            one = jnp.float32(1.0)
            zero = jnp.float32(0.0)
            for h in (0, 1):
                pltpu.sync_copy(
                    sim_hbm.at[b, pl.ds(h * PH, PH), pl.ds(cb, CB)], in_v
                )
                t16s = [run_v[g, _K - 1, :] for g in range(NG)]

                def mask_body(r, carry2):
                    for g in range(NG):
                        gs = pl.ds(g * L, L)
                        row = in_v[r, gs]
                        in_v[r, gs] = jnp.where(row >= t16s[g], one, zero)
                    return carry2

                lax.fori_loop(0, PH, mask_body, 0)
                pltpu.sync_copy(
                    in_v, mask_hbm.at[b, pl.ds(h * PH, PH), pl.ds(cb, CB)]
                )
            return carry

        lax.fori_loop(0, tasks_per_w, task_body, 0)

    scores3, mask = sc_body(similarity_matrix)
    return scores3.reshape(B, C), mask


# speedup vs baseline: 3.8291x; 3.8291x over previous
import jax
import jax.numpy as jnp
from jax.experimental import pallas as pl

_K = 16
_BB = 4


def _batcher_pairs(n):
    pairs = []

    def merge(lo, m, r):
        step = r * 2
        if step < m:
            merge(lo, m, step)
            merge(lo + r, m, step)
            for i in range(lo + r, lo + m - r, step):
                pairs.append((i, i + r))
        else:
            pairs.append((lo, lo + r))

    def sort(lo, m):
        if m > 1:
            h = m // 2
            sort(lo, h)
            sort(lo + h, h)
            merge(lo, m, 1)

    sort(0, n)
    return pairs


_SORT16 = _batcher_pairs(_K)


def _sort_desc(v):
    v = list(v)
    for i, j in _SORT16:
        hi = jnp.maximum(v[i], v[j])
        lo = jnp.minimum(v[i], v[j])
        v[i], v[j] = hi, lo
    return v


def _bitonic_resort(v):
    n = len(v)
    if n == 1:
        return v
    h = n // 2
    hi = [jnp.maximum(v[i], v[i + h]) for i in range(h)]
    lo = [jnp.minimum(v[i], v[i + h]) for i in range(h)]
    return _bitonic_resort(hi) + _bitonic_resort(lo)


def _merge_top16(a, b):
    merged = [jnp.maximum(a[i], b[_K - 1 - i]) for i in range(_K)]
    return _bitonic_resort(merged)


def _topk_body(x_ref, scores_ref, mask_ref):
    P = x_ref.shape[1]
    C = x_ref.shape[2]
    CB = 128
    n_chunks = P // (8 * _K)

    for bi in range(_BB):
        for c in range(C // CB):
            csl = slice(c * CB, (c + 1) * CB)
            run = None
            for s in range(n_chunks):
                base = s * 8 * _K
                tiles = [
                    x_ref[bi, base + 8 * i : base + 8 * (i + 1), csl]
                    for i in range(_K)
                ]
                tiles = _sort_desc(tiles)
                run = tiles if run is None else _merge_top16(run, tiles)
            for d in (1, 2, 4):
                rolled = [
                    jnp.roll(run[_K - 1 - i], d, axis=0) for i in range(_K)
                ]
                run = _bitonic_resort(
                    [jnp.maximum(run[i], rolled[i]) for i in range(_K)]
                )
            scores_ref[bi, 0:1, csl] = run[0][0:1]
            t16 = run[_K - 1]
            one = jnp.float32(1.0)
            zero = jnp.float32(0.0)
            for s in range(P // 8):
                xa = x_ref[bi, 8 * s : 8 * (s + 1), csl]
                mask_ref[bi, 8 * s : 8 * (s + 1), csl] = jnp.where(
                    xa >= t16, one, zero
                )


def kernel(similarity_matrix):
    B, P, C = similarity_matrix.shape
    scores3, mask = pl.pallas_call(
        _topk_body,
        grid=(B // _BB,),
        in_specs=[pl.BlockSpec((_BB, P, C), lambda b: (b, 0, 0))],
        out_specs=[
            pl.BlockSpec((_BB, 1, C), lambda b: (b, 0, 0)),
            pl.BlockSpec((_BB, P, C), lambda b: (b, 0, 0)),
        ],
        out_shape=[
            jax.ShapeDtypeStruct((B, 1, C), jnp.float32),
            jax.ShapeDtypeStruct((B, P, C), jnp.float32),
        ],
    )(similarity_matrix)
    return scores3.reshape(B, C), mask
